# baseline (speedup 1.0000x reference)
import functools

import jax
import jax.numpy as jnp
from jax.experimental import pallas as pl
from jax.experimental.pallas import tpu as pltpu


def _ffwd_bf16_kernel(x_ref, w1_ref, b1_ref, w2_ref, b2_ref, o_ref, *,
                      negative_slope):
    x = x_ref[...].astype(jnp.bfloat16)
    h = jnp.dot(x, w1_ref[...], preferred_element_type=jnp.float32)
    h += b1_ref[...]
    h = jnp.maximum(h, negative_slope * h)
    out = jnp.dot(h.astype(jnp.bfloat16), w2_ref[...],
                  preferred_element_type=jnp.float32)
    o_ref[...] = (out + b2_ref[...]).astype(o_ref.dtype)


def kernel(x, w1, b1, w2, b2, *, negative_slope=0.01, tm=512):
    B, T, E = x.shape
    H = w1.shape[1]
    M = B * T
    out_dtype = x.dtype

    x2d = x.reshape(M, E)
    w1b = w1.astype(jnp.bfloat16)
    w2b = w2.astype(jnp.bfloat16)
    b1_2d = b1.reshape(1, H).astype(jnp.float32)
    b2_2d = b2.reshape(1, E).astype(jnp.float32)

    tm = min(tm, M)
    gm = pl.cdiv(M, tm)

    cost = pl.CostEstimate(
        flops=4 * M * E * H,
        transcendentals=0,
        bytes_accessed=M * E * 8 + 2 * E * H * 2 + (H + E) * 4,
    )

    out2d = pl.pallas_call(
        functools.partial(_ffwd_bf16_kernel, negative_slope=negative_slope),
        out_shape=jax.ShapeDtypeStruct((M, E), out_dtype),
        grid=(gm,),
        in_specs=[
            pl.BlockSpec((tm, E), lambda i: (i, 0)),
            pl.BlockSpec((E, H), lambda i: (0, 0),
                         pipeline_mode=pl.Buffered(1)),
            pl.BlockSpec((1, H), lambda i: (0, 0),
                         pipeline_mode=pl.Buffered(1)),
            pl.BlockSpec((H, E), lambda i: (0, 0),
                         pipeline_mode=pl.Buffered(1)),
            pl.BlockSpec((1, E), lambda i: (0, 0),
                         pipeline_mode=pl.Buffered(1)),
        ],
        out_specs=pl.BlockSpec((tm, E), lambda i: (i, 0)),
        compiler_params=pltpu.CompilerParams(
            dimension_semantics=("parallel",),
            vmem_limit_bytes=int(48 << 20),
        ),
        cost_estimate=cost,
    )(x2d, w1b, b1_2d, w2b, b2_2d)

    return out2d.reshape(B, T, E)


# speedup vs baseline: 1.0003x; 1.0003x over previous
import functools

import jax
import jax.numpy as jnp
from jax.experimental import pallas as pl
from jax.experimental.pallas import tpu as pltpu


def _ffwd_kernel(x_ref, w1_ref, b1_ref, w2_ref, b2_ref, o_ref, *,
                 negative_slope):
    x = x_ref[...].astype(jnp.bfloat16)
    h = jnp.dot(x, w1_ref[...],
                preferred_element_type=jnp.float32).astype(jnp.bfloat16)
    h += b1_ref[...]
    h = jnp.maximum(h, jnp.bfloat16(negative_slope) * h)
    out = jnp.dot(h, w2_ref[...], preferred_element_type=jnp.float32)
    o_ref[...] = (out + b2_ref[...]).astype(o_ref.dtype)


def kernel(x, w1, b1, w2, b2, *, negative_slope=0.01, tm=2048):
    B, T, E = x.shape
    H = w1.shape[1]
    M = B * T
    out_dtype = x.dtype

    x2d = x.reshape(M, E)
    w1b = w1.astype(jnp.bfloat16)
    w2b = w2.astype(jnp.bfloat16)
    b1_2d = b1.reshape(1, H).astype(jnp.bfloat16)
    b2_2d = b2.reshape(1, E).astype(jnp.float32)

    tm = min(tm, M)
    gm = pl.cdiv(M, tm)

    cost = pl.CostEstimate(
        flops=4 * M * E * H,
        transcendentals=0,
        bytes_accessed=M * E * 8 + 2 * E * H * 2 + (H + E) * 4,
    )

    out2d = pl.pallas_call(
        functools.partial(_ffwd_kernel, negative_slope=negative_slope),
        out_shape=jax.ShapeDtypeStruct((M, E), out_dtype),
        grid=(gm,),
        in_specs=[
            pl.BlockSpec((tm, E), lambda i: (i, 0)),
            pl.BlockSpec((E, H), lambda i: (0, 0),
                         pipeline_mode=pl.Buffered(1)),
            pl.BlockSpec((1, H), lambda i: (0, 0),
                         pipeline_mode=pl.Buffered(1)),
            pl.BlockSpec((H, E), lambda i: (0, 0),
                         pipeline_mode=pl.Buffered(1)),
            pl.BlockSpec((1, E), lambda i: (0, 0),
                         pipeline_mode=pl.Buffered(1)),
        ],
        out_specs=pl.BlockSpec((tm, E), lambda i: (i, 0)),
        compiler_params=pltpu.CompilerParams(
            dimension_semantics=("parallel",),
            vmem_limit_bytes=int(57 << 20),
        ),
        cost_estimate=cost,
    )(x2d, w1b, b1_2d, w2b, b2_2d)

    return out2d.reshape(B, T, E)
